# baseline (speedup 1.0000x reference)
import functools

import numpy as np

import jax
import jax.numpy as jnp
from jax.experimental import pallas as pl
from jax.experimental.pallas import tpu as pltpu
from jax.experimental.pallas import tpu_sc as plsc

T = 2048
D = 1024
E = 8
K = 2
I = 512
TR = 512
SP = T * K + E * TR
NT = SP // TR
SPF = SP + T
NTF = SPF // TR
NW = 32
CHUNK = T // NW
CB = 32
G = 16


def _routing_kernel(x_ref, wg_ref, u_ref, qt_ref, qred_ref, w8_ref, l16_ref,
                    mlt_ref, pos1_ref, pos2_ref, w1_ref, w2_ref, eot_ref):
    x = x_ref[...]
    logits = jnp.dot(x, wg_ref[...], preferred_element_type=jnp.float32)
    probs = jax.nn.softmax(logits, axis=-1)
    ei = jax.lax.broadcasted_iota(jnp.int32, (T, E), 1)
    v1 = jnp.max(probs, axis=1, keepdims=True)
    i1 = jnp.min(jnp.where(probs == v1, ei, E), axis=1, keepdims=True)
    pr2 = jnp.where(ei == i1, -jnp.inf, probs)
    v2 = jnp.max(pr2, axis=1, keepdims=True)
    i2 = jnp.min(jnp.where(pr2 == v2, ei, E), axis=1, keepdims=True)
    den = v1 + v2
    w1_ref[...] = jnp.broadcast_to(v1 / den, (T, 16))
    w2_ref[...] = jnp.broadcast_to(v2 / den, (T, 16))

    lane = jax.lax.broadcasted_iota(jnp.int32, (G, E * 128), 1)
    ef = (lane % E).astype(jnp.float32)

    def prefix(idx):
        idr = idx.reshape(G, 128).astype(jnp.float32)
        idrep = jnp.dot(idr, qt_ref[...], preferred_element_type=jnp.float32)
        a = (idrep == ef).astype(jnp.float32)
        c = jnp.dot(a.astype(jnp.bfloat16), u_ref[...],
                    preferred_element_type=jnp.float32)
        rt = c[:, (128 - 1) * E:]
        off = jnp.dot(l16_ref[...], rt, preferred_element_type=jnp.float32)
        tot = off[G - 1:G, :] + rt[G - 1:G, :]
        return a, c, off, tot

    a1, c1, off1, tot1 = prefix(i1)
    a2, c2, off2, tot2 = prefix(i2)
    counts = tot1 + tot2
    cp = jnp.ceil(counts / TR) * TR
    base = jnp.dot(cp, mlt_ref[...], preferred_element_type=jnp.float32)

    cum = base + cp
    tt = (jax.lax.broadcasted_iota(jnp.int32, (32, E), 0) * TR).astype(
        jnp.float32)
    cmp = (tt >= jnp.broadcast_to(cum, (32, E))).astype(jnp.int32)
    eotc = jnp.sum(cmp, axis=1, keepdims=True)
    eot_ref[...] = jnp.broadcast_to(eotc, (32, 128))

    hi = jax.lax.Precision.HIGHEST
    b1 = jnp.dot(base, w8_ref[...], preferred_element_type=jnp.float32,
                 precision=hi)
    b2 = jnp.dot(base + tot1, w8_ref[...], preferred_element_type=jnp.float32,
                 precision=hi)
    o1 = jnp.dot(off1, w8_ref[...], preferred_element_type=jnp.float32,
                 precision=hi)
    o2 = jnp.dot(off2, w8_ref[...], preferred_element_type=jnp.float32,
                 precision=hi)
    x1 = a1 * (c1 - 1.0 + o1 + jnp.broadcast_to(b1, (G, E * 128)))
    x2 = a2 * (c2 - 1.0 + o2 + jnp.broadcast_to(b2, (G, E * 128)))
    p1 = jnp.dot(x1, qred_ref[...], preferred_element_type=jnp.float32,
                 precision=hi)
    p2 = jnp.dot(x2, qred_ref[...], preferred_element_type=jnp.float32,
                 precision=hi)
    pos1_ref[...] = p1.astype(jnp.int32)
    pos2_ref[...] = p2.astype(jnp.int32)


def _sort_consts():
    l = np.arange(128)
    e = np.arange(E)
    c_l, c_e = np.divmod(np.arange(E * 128), E)
    qt = (l[:, None] == c_l[None, :]).astype(np.float32)
    u = ((c_e[:, None] == c_e[None, :])
         & (c_l[:, None] <= c_l[None, :])).astype(np.float32)
    qred = (c_l[:, None] == l[None, :]).astype(np.float32)
    w8 = (e[:, None] == c_e[None, :]).astype(np.float32)
    l16 = (np.arange(G)[None, :] < np.arange(G)[:, None]).astype(np.float32)
    mlt = (e[:, None] < e[None, :]).astype(np.float32)
    return (jnp.asarray(u, jnp.bfloat16), jnp.asarray(qt),
            jnp.asarray(qred), jnp.asarray(w8), jnp.asarray(l16),
            jnp.asarray(mlt))


def _routing(x, w_gate):
    u, qt, qred, w8, l16, mlt = _sort_consts()
    full = lambda s: pl.BlockSpec(s, lambda: (0,) * len(s))
    outs = pl.pallas_call(
        _routing_kernel,
        in_specs=[full((T, D)), full((D, E)), full((E * 128, E * 128)),
                  full((128, E * 128)), full((E * 128, 128)),
                  full((E, E * 128)), full((G, G)), full((E, E))],
        out_specs=[full((G, 128)), full((G, 128)), full((T, 16)),
                   full((T, 16)), full((32, 128))],
        out_shape=[
            jax.ShapeDtypeStruct((G, 128), jnp.int32),
            jax.ShapeDtypeStruct((G, 128), jnp.int32),
            jax.ShapeDtypeStruct((T, 16), jnp.float32),
            jax.ShapeDtypeStruct((T, 16), jnp.float32),
            jax.ShapeDtypeStruct((32, 128), jnp.int32),
        ],
    )(x, w_gate, u, qt, qred, w8, l16, mlt)
    return outs


@functools.cache
def _make_dispatch():
    mesh = plsc.VectorSubcoreMesh(core_axis_name="c", subcore_axis_name="s")

    @functools.partial(
        pl.kernel, mesh=mesh,
        out_type=jax.ShapeDtypeStruct((SPF, D), jnp.float32),
        scratch_types=[
            pltpu.VMEM((CHUNK,), jnp.int32),
            pltpu.VMEM((CHUNK,), jnp.int32),
            pltpu.VMEM((CHUNK, D), jnp.float32),
            pltpu.SemaphoreType.DMA,
        ],
    )
    def _dispatch(x_hbm, p1_hbm, p2_hbm, xg_hbm, i1_v, i2_v, rows_v, sem):
        wid = jax.lax.axis_index("s") * 2 + jax.lax.axis_index("c")
        base = wid * CHUNK
        c1 = pltpu.async_copy(p1_hbm.at[pl.ds(base, CHUNK)], i1_v, sem)
        c2 = pltpu.async_copy(p2_hbm.at[pl.ds(base, CHUNK)], i2_v, sem)
        c3 = pltpu.async_copy(x_hbm.at[pl.ds(base, CHUNK)], rows_v, sem)
        c1.wait()
        c2.wait()
        c3.wait()
        s1 = pltpu.async_copy(rows_v, xg_hbm.at[i1_v], sem)
        s2 = pltpu.async_copy(rows_v, xg_hbm.at[i2_v], sem)
        s3 = pltpu.async_copy(rows_v, xg_hbm.at[pl.ds(SP + base, CHUNK)], sem)
        s1.wait()
        s2.wait()
        s3.wait()

    return _dispatch


def _gmm_kernel(eot_ref, xg_ref, wgu_ref, wd_ref, yg_ref):
    e = eot_ref[pl.program_id(0)]

    @pl.when(e <= E)
    def _():
        xb = xg_ref[...].astype(jnp.bfloat16)
        gu = jnp.dot(xb, wgu_ref[0], preferred_element_type=jnp.float32)
        g = gu[:, :I]
        u = gu[:, I:]
        h = (g * jax.lax.logistic(g)) * u
        yg_ref[...] = jnp.dot(h.astype(jnp.bfloat16), wd_ref[0],
                              preferred_element_type=jnp.float32)


def _gmm(eot, xg, wgu, wd):
    wexp = lambda i, eot_ref: (jnp.minimum(eot_ref[i], E), 0, 0)
    grid_spec = pltpu.PrefetchScalarGridSpec(
        num_scalar_prefetch=1,
        grid=(NTF,),
        in_specs=[
            pl.BlockSpec((TR, D), lambda i, eot_ref: (i, 0)),
            pl.BlockSpec((1, D, 2 * I), wexp),
            pl.BlockSpec((1, I, D), wexp),
        ],
        out_specs=pl.BlockSpec((TR, D), lambda i, eot_ref: (i, 0)),
    )
    return pl.pallas_call(
        _gmm_kernel,
        grid_spec=grid_spec,
        out_shape=jax.ShapeDtypeStruct((SPF, D), jnp.float32),
        compiler_params=pltpu.CompilerParams(
            dimension_semantics=("arbitrary",),
        ),
    )(eot, xg, wgu, wd)


def _shared_kernel(x_ref, sgu_ref, sd_ref, o_ref):
    xb = x_ref[...].astype(jnp.bfloat16)
    gu = jnp.dot(xb, sgu_ref[...], preferred_element_type=jnp.float32)
    g = gu[:, :I]
    u = gu[:, I:]
    h = (g * jax.lax.logistic(g)) * u
    o_ref[...] = jnp.dot(h.astype(jnp.bfloat16), sd_ref[...],
                         preferred_element_type=jnp.float32)


def _shared(x, sgu, sd):
    return pl.pallas_call(
        _shared_kernel,
        grid=(2,),
        in_specs=[
            pl.BlockSpec((T // 2, D), lambda m: (m, 0)),
            pl.BlockSpec((D, 2 * I), lambda m: (0, 0)),
            pl.BlockSpec((I, D), lambda m: (0, 0)),
        ],
        out_specs=pl.BlockSpec((T // 2, D), lambda m: (m, 0)),
        out_shape=jax.ShapeDtypeStruct((T, D), jnp.float32),
    )(x, sgu, sd)


@functools.cache
def _make_combine():
    mesh = plsc.VectorSubcoreMesh(core_axis_name="c", subcore_axis_name="s")

    @functools.partial(
        pl.kernel, mesh=mesh,
        out_type=jax.ShapeDtypeStruct((T, D), jnp.float32),
        scratch_types=[
            pltpu.VMEM((CB,), jnp.int32),
            pltpu.VMEM((CB,), jnp.int32),
            pltpu.VMEM((CB, D), jnp.float32),
            pltpu.VMEM((CB, D), jnp.float32),
            pltpu.VMEM((CB, D), jnp.float32),
            pltpu.VMEM((CB, 16), jnp.float32),
            pltpu.VMEM((CB, 16), jnp.float32),
            pltpu.SemaphoreType.DMA,
        ],
    )
    def _combine(yg_hbm, p1_hbm, p2_hbm, w1_hbm, w2_hbm, out_hbm,
                 i1_v, i2_v, y1_v, y2_v, acc_v, w1_v, w2_v, sem):
        wid = jax.lax.axis_index("s") * 2 + jax.lax.axis_index("c")
        for b in range(CHUNK // CB):
            base = wid * CHUNK + b * CB
            c1 = pltpu.async_copy(p1_hbm.at[pl.ds(base, CB)], i1_v, sem)
            c2 = pltpu.async_copy(p2_hbm.at[pl.ds(base, CB)], i2_v, sem)
            c3 = pltpu.async_copy(yg_hbm.at[pl.ds(SP + base, CB)], acc_v, sem)
            c4 = pltpu.async_copy(w1_hbm.at[pl.ds(base, CB)], w1_v, sem)
            c5 = pltpu.async_copy(w2_hbm.at[pl.ds(base, CB)], w2_v, sem)
            c1.wait()
            c2.wait()
            g1 = pltpu.async_copy(yg_hbm.at[i1_v], y1_v, sem)
            g2 = pltpu.async_copy(yg_hbm.at[i2_v], y2_v, sem)
            c3.wait()
            c4.wait()
            c5.wait()
            g1.wait()
            g2.wait()

            def body(t, carry):
                w1t = w1_v[t, :]
                w2t = w2_v[t, :]
                for j in range(D // 16):
                    sl = pl.ds(j * 16, 16)
                    acc_v[t, sl] = (acc_v[t, sl] + w1t * y1_v[t, sl]
                                    + w2t * y2_v[t, sl])
                return carry

            jax.lax.fori_loop(0, CB, body, 0)
            pltpu.sync_copy(acc_v, out_hbm.at[pl.ds(base, CB)])

    return _combine


def kernel(hidden_states, w_gate, w_gate_up, w_down, shared_gate_up,
           shared_down):
    x = hidden_states
    wgu = jnp.concatenate([w_gate_up, shared_gate_up[None]],
                          axis=0).astype(jnp.bfloat16)
    wd = jnp.concatenate([w_down, shared_down[None]],
                         axis=0).astype(jnp.bfloat16)

    pos1_2d, pos2_2d, w1rep, w2rep, eot2d = _routing(x, w_gate)
    pos1 = pos1_2d.reshape(T)
    pos2 = pos2_2d.reshape(T)
    eot = eot2d[:NT, 0]
    eot_full = jnp.concatenate([
        jnp.where(eot >= E, E + 1, eot),
        jnp.full((T // TR,), E, jnp.int32),
    ])

    xg = _make_dispatch()(x, pos1, pos2)
    yg = _gmm(eot_full, xg, wgu, wd)
    out = _make_combine()(yg, pos1, pos2, w1rep, w2rep)
    return out


# speedup vs baseline: 1.1357x; 1.1357x over previous
import functools

import numpy as np

import jax
import jax.numpy as jnp
from jax.experimental import pallas as pl
from jax.experimental.pallas import tpu as pltpu
from jax.experimental.pallas import tpu_sc as plsc

T = 2048
D = 1024
E = 8
K = 2
I = 512
TR = 512
SP = T * K + E * TR
NT = SP // TR
SPF = SP + T
NTF = SPF // TR
NW = 32
CHUNK = T // NW
CB = 32
G = 16


def _routing_kernel(x_ref, wg_ref, u_ref, qt_ref, qred_ref, w8_ref, l16_ref,
                    mlt_ref, pos1_ref, pos2_ref, w1_ref, w2_ref, eot_ref):
    x = x_ref[...]
    logits = jnp.dot(x, wg_ref[...], preferred_element_type=jnp.float32)
    probs = jax.nn.softmax(logits, axis=-1)
    ei = jax.lax.broadcasted_iota(jnp.int32, (T, E), 1)
    v1 = jnp.max(probs, axis=1, keepdims=True)
    i1 = jnp.min(jnp.where(probs == v1, ei, E), axis=1, keepdims=True)
    pr2 = jnp.where(ei == i1, -jnp.inf, probs)
    v2 = jnp.max(pr2, axis=1, keepdims=True)
    i2 = jnp.min(jnp.where(pr2 == v2, ei, E), axis=1, keepdims=True)
    den = v1 + v2
    w1_ref[...] = jnp.broadcast_to(v1 / den, (T, 16))
    w2_ref[...] = jnp.broadcast_to(v2 / den, (T, 16))

    lane = jax.lax.broadcasted_iota(jnp.int32, (G, E * 128), 1)
    ef = (lane % E).astype(jnp.float32)

    def prefix(idx):
        idr = idx.reshape(G, 128).astype(jnp.float32)
        idrep = jnp.dot(idr, qt_ref[...], preferred_element_type=jnp.float32)
        a = (idrep == ef).astype(jnp.float32)
        c = jnp.dot(a.astype(jnp.bfloat16), u_ref[...],
                    preferred_element_type=jnp.float32)
        rt = c[:, (128 - 1) * E:]
        off = jnp.dot(l16_ref[...], rt, preferred_element_type=jnp.float32)
        tot = off[G - 1:G, :] + rt[G - 1:G, :]
        return a, c, off, tot

    a1, c1, off1, tot1 = prefix(i1)
    a2, c2, off2, tot2 = prefix(i2)
    counts = tot1 + tot2
    cp = jnp.ceil(counts / TR) * TR
    base = jnp.dot(cp, mlt_ref[...], preferred_element_type=jnp.float32)

    cum = base + cp
    tt = (jax.lax.broadcasted_iota(jnp.int32, (32, E), 0) * TR).astype(
        jnp.float32)
    cmp = (tt >= jnp.broadcast_to(cum, (32, E))).astype(jnp.int32)
    eotc = jnp.sum(cmp, axis=1, keepdims=True)
    eot_ref[...] = jnp.broadcast_to(eotc, (32, 128))

    hi = jax.lax.Precision.HIGHEST
    b1 = jnp.dot(base, w8_ref[...], preferred_element_type=jnp.float32,
                 precision=hi)
    b2 = jnp.dot(base + tot1, w8_ref[...], preferred_element_type=jnp.float32,
                 precision=hi)
    o1 = jnp.dot(off1, w8_ref[...], preferred_element_type=jnp.float32,
                 precision=hi)
    o2 = jnp.dot(off2, w8_ref[...], preferred_element_type=jnp.float32,
                 precision=hi)
    x1 = a1 * (c1 - 1.0 + o1 + jnp.broadcast_to(b1, (G, E * 128)))
    x2 = a2 * (c2 - 1.0 + o2 + jnp.broadcast_to(b2, (G, E * 128)))
    p1 = jnp.dot(x1, qred_ref[...], preferred_element_type=jnp.float32,
                 precision=hi)
    p2 = jnp.dot(x2, qred_ref[...], preferred_element_type=jnp.float32,
                 precision=hi)
    pos1_ref[...] = p1.astype(jnp.int32)
    pos2_ref[...] = p2.astype(jnp.int32)


def _sort_consts():
    l = np.arange(128)
    e = np.arange(E)
    c_l, c_e = np.divmod(np.arange(E * 128), E)
    qt = (l[:, None] == c_l[None, :]).astype(np.float32)
    u = ((c_e[:, None] == c_e[None, :])
         & (c_l[:, None] <= c_l[None, :])).astype(np.float32)
    qred = (c_l[:, None] == l[None, :]).astype(np.float32)
    w8 = (e[:, None] == c_e[None, :]).astype(np.float32)
    l16 = (np.arange(G)[None, :] < np.arange(G)[:, None]).astype(np.float32)
    mlt = (e[:, None] < e[None, :]).astype(np.float32)
    return (jnp.asarray(u, jnp.bfloat16), jnp.asarray(qt),
            jnp.asarray(qred), jnp.asarray(w8), jnp.asarray(l16),
            jnp.asarray(mlt))


def _routing(x, w_gate):
    u, qt, qred, w8, l16, mlt = _sort_consts()
    full = lambda s: pl.BlockSpec(s, lambda: (0,) * len(s))
    outs = pl.pallas_call(
        _routing_kernel,
        in_specs=[full((T, D)), full((D, E)), full((E * 128, E * 128)),
                  full((128, E * 128)), full((E * 128, 128)),
                  full((E, E * 128)), full((G, G)), full((E, E))],
        out_specs=[full((G, 128)), full((G, 128)), full((T, 16)),
                   full((T, 16)), full((32, 128))],
        out_shape=[
            jax.ShapeDtypeStruct((G, 128), jnp.int32),
            jax.ShapeDtypeStruct((G, 128), jnp.int32),
            jax.ShapeDtypeStruct((T, 16), jnp.float32),
            jax.ShapeDtypeStruct((T, 16), jnp.float32),
            jax.ShapeDtypeStruct((32, 128), jnp.int32),
        ],
    )(x, w_gate, u, qt, qred, w8, l16, mlt)
    return outs


@functools.cache
def _make_dispatch():
    mesh = plsc.VectorSubcoreMesh(core_axis_name="c", subcore_axis_name="s")

    @functools.partial(
        pl.kernel, mesh=mesh,
        out_type=jax.ShapeDtypeStruct((SP, D), jnp.float32),
        scratch_types=[
            pltpu.VMEM((CHUNK,), jnp.int32),
            pltpu.VMEM((CHUNK,), jnp.int32),
            pltpu.VMEM((CHUNK, D), jnp.float32),
            pltpu.SemaphoreType.DMA,
        ],
    )
    def _dispatch(x_hbm, p1_hbm, p2_hbm, xg_hbm, i1_v, i2_v, rows_v, sem):
        wid = jax.lax.axis_index("s") * 2 + jax.lax.axis_index("c")
        base = wid * CHUNK
        c1 = pltpu.async_copy(p1_hbm.at[pl.ds(base, CHUNK)], i1_v, sem)
        c2 = pltpu.async_copy(p2_hbm.at[pl.ds(base, CHUNK)], i2_v, sem)
        c3 = pltpu.async_copy(x_hbm.at[pl.ds(base, CHUNK)], rows_v, sem)
        c1.wait()
        c2.wait()
        c3.wait()
        s1 = pltpu.async_copy(rows_v, xg_hbm.at[i1_v], sem)
        s2 = pltpu.async_copy(rows_v, xg_hbm.at[i2_v], sem)
        s1.wait()
        s2.wait()

    return _dispatch


def _gmm_kernel(eot_ref, xg_ref, wgu_ref, wd_ref, yg_ref):
    e = eot_ref[pl.program_id(0)]

    @pl.when(e < E)
    def _():
        xb = xg_ref[...].astype(jnp.bfloat16)
        gu = jnp.dot(xb, wgu_ref[0], preferred_element_type=jnp.float32)
        g = gu[:, :I]
        u = gu[:, I:]
        h = (g * jax.lax.logistic(g)) * u
        yg_ref[...] = jnp.dot(h.astype(jnp.bfloat16), wd_ref[0],
                              preferred_element_type=jnp.float32)


def _gmm(eot, xg, wgu, wd):
    wexp = lambda i, eot_ref: (jnp.minimum(eot_ref[i], E - 1), 0, 0)
    grid_spec = pltpu.PrefetchScalarGridSpec(
        num_scalar_prefetch=1,
        grid=(NT,),
        in_specs=[
            pl.BlockSpec((TR, D), lambda i, eot_ref: (i, 0)),
            pl.BlockSpec((1, D, 2 * I), wexp),
            pl.BlockSpec((1, I, D), wexp),
        ],
        out_specs=pl.BlockSpec((TR, D), lambda i, eot_ref: (i, 0)),
    )
    return pl.pallas_call(
        _gmm_kernel,
        grid_spec=grid_spec,
        out_shape=jax.ShapeDtypeStruct((SP, D), jnp.float32),
        compiler_params=pltpu.CompilerParams(
            dimension_semantics=("arbitrary",),
        ),
    )(eot, xg, wgu, wd)


def _shared_kernel(x_ref, sgu_ref, sd_ref, o_ref):
    xb = x_ref[...].astype(jnp.bfloat16)
    gu = jnp.dot(xb, sgu_ref[...], preferred_element_type=jnp.float32)
    g = gu[:, :I]
    u = gu[:, I:]
    h = (g * jax.lax.logistic(g)) * u
    o_ref[...] = jnp.dot(h.astype(jnp.bfloat16), sd_ref[...],
                         preferred_element_type=jnp.float32)


def _shared(x, sgu, sd):
    return pl.pallas_call(
        _shared_kernel,
        grid=(2,),
        in_specs=[
            pl.BlockSpec((T // 2, D), lambda m: (m, 0)),
            pl.BlockSpec((D, 2 * I), lambda m: (0, 0)),
            pl.BlockSpec((I, D), lambda m: (0, 0)),
        ],
        out_specs=pl.BlockSpec((T // 2, D), lambda m: (m, 0)),
        out_shape=jax.ShapeDtypeStruct((T, D), jnp.float32),
    )(x, sgu, sd)


@functools.cache
def _make_combine():
    mesh = plsc.VectorSubcoreMesh(core_axis_name="c", subcore_axis_name="s")

    @functools.partial(
        pl.kernel, mesh=mesh,
        out_type=jax.ShapeDtypeStruct((T, D), jnp.float32),
        scratch_types=[
            pltpu.VMEM((CB,), jnp.int32),
            pltpu.VMEM((CB,), jnp.int32),
            pltpu.VMEM((CB, D), jnp.float32),
            pltpu.VMEM((CB, D), jnp.float32),
            pltpu.VMEM((CB, D), jnp.float32),
            pltpu.VMEM((CB, 16), jnp.float32),
            pltpu.VMEM((CB, 16), jnp.float32),
            pltpu.SemaphoreType.DMA,
        ],
    )
    def _combine(yg_hbm, sh_hbm, p1_hbm, p2_hbm, w1_hbm, w2_hbm, out_hbm,
                 i1_v, i2_v, y1_v, y2_v, acc_v, w1_v, w2_v, sem):
        wid = jax.lax.axis_index("s") * 2 + jax.lax.axis_index("c")
        for b in range(CHUNK // CB):
            base = wid * CHUNK + b * CB
            c1 = pltpu.async_copy(p1_hbm.at[pl.ds(base, CB)], i1_v, sem)
            c2 = pltpu.async_copy(p2_hbm.at[pl.ds(base, CB)], i2_v, sem)
            c3 = pltpu.async_copy(sh_hbm.at[pl.ds(base, CB)], acc_v, sem)
            c4 = pltpu.async_copy(w1_hbm.at[pl.ds(base, CB)], w1_v, sem)
            c5 = pltpu.async_copy(w2_hbm.at[pl.ds(base, CB)], w2_v, sem)
            c1.wait()
            c2.wait()
            g1 = pltpu.async_copy(yg_hbm.at[i1_v], y1_v, sem)
            g2 = pltpu.async_copy(yg_hbm.at[i2_v], y2_v, sem)
            c3.wait()
            c4.wait()
            c5.wait()
            g1.wait()
            g2.wait()

            def body(t, carry):
                w1t = w1_v[t, :]
                w2t = w2_v[t, :]
                for j in range(D // 16):
                    sl = pl.ds(j * 16, 16)
                    acc_v[t, sl] = (acc_v[t, sl] + w1t * y1_v[t, sl]
                                    + w2t * y2_v[t, sl])
                return carry

            jax.lax.fori_loop(0, CB, body, 0)
            pltpu.sync_copy(acc_v, out_hbm.at[pl.ds(base, CB)])

    return _combine


def kernel(hidden_states, w_gate, w_gate_up, w_down, shared_gate_up,
           shared_down):
    x = hidden_states
    wgu = w_gate_up.astype(jnp.bfloat16)
    wd = w_down.astype(jnp.bfloat16)
    sgu = shared_gate_up.astype(jnp.bfloat16)
    sd = shared_down.astype(jnp.bfloat16)

    pos1_2d, pos2_2d, w1rep, w2rep, eot2d = _routing(x, w_gate)
    pos1 = pos1_2d.reshape(T)
    pos2 = pos2_2d.reshape(T)
    eot = eot2d[:NT, 0]

    xg = _make_dispatch()(x, pos1, pos2)
    sh = _shared(x, sgu, sd)
    yg = _gmm(eot, xg, wgu, wd)
    out = _make_combine()(yg, sh, pos1, pos2, w1rep, w2rep)
    return out


# speedup vs baseline: 1.1359x; 1.0002x over previous
import functools

import numpy as np

import jax
import jax.numpy as jnp
from jax.experimental import pallas as pl
from jax.experimental.pallas import tpu as pltpu
from jax.experimental.pallas import tpu_sc as plsc

T = 2048
D = 1024
E = 8
K = 2
I = 512
TR = 512
SP = T * K + E * TR
NT = SP // TR
NW = 32
CHUNK = T // NW
CB = 32
G = 16


def _routing_kernel(x_ref, wg_ref, u_ref, qt_ref, qred_ref, w8_ref, l16_ref,
                    mlt_ref, pos1_ref, pos2_ref, w1_ref, w2_ref, eot_ref):
    x = x_ref[...]
    logits = jnp.dot(x, wg_ref[...], preferred_element_type=jnp.float32)
    probs = jax.nn.softmax(logits, axis=-1)
    ei = jax.lax.broadcasted_iota(jnp.int32, (T, E), 1)
    v1 = jnp.max(probs, axis=1, keepdims=True)
    i1 = jnp.min(jnp.where(probs == v1, ei, E), axis=1, keepdims=True)
    pr2 = jnp.where(ei == i1, -jnp.inf, probs)
    v2 = jnp.max(pr2, axis=1, keepdims=True)
    i2 = jnp.min(jnp.where(pr2 == v2, ei, E), axis=1, keepdims=True)
    den = v1 + v2
    w1_ref[...] = jnp.broadcast_to(v1 / den, (T, 16))
    w2_ref[...] = jnp.broadcast_to(v2 / den, (T, 16))

    lane = jax.lax.broadcasted_iota(jnp.int32, (G, E * 128), 1)
    ef = (lane % E).astype(jnp.float32)

    def prefix(idx):
        idr = idx.reshape(G, 128).astype(jnp.float32)
        idrep = jnp.dot(idr, qt_ref[...], preferred_element_type=jnp.float32)
        a = (idrep == ef).astype(jnp.float32)
        c = jnp.dot(a.astype(jnp.bfloat16), u_ref[...],
                    preferred_element_type=jnp.float32)
        rt = c[:, (128 - 1) * E:]
        off = jnp.dot(l16_ref[...], rt, preferred_element_type=jnp.float32)
        tot = off[G - 1:G, :] + rt[G - 1:G, :]
        return a, c, off, tot

    a1, c1, off1, tot1 = prefix(i1)
    a2, c2, off2, tot2 = prefix(i2)
    counts = tot1 + tot2
    cp = jnp.ceil(counts / TR) * TR
    base = jnp.dot(cp, mlt_ref[...], preferred_element_type=jnp.float32)

    cum = base + cp
    tt = (jax.lax.broadcasted_iota(jnp.int32, (32, E), 0) * TR).astype(
        jnp.float32)
    cmp = (tt >= jnp.broadcast_to(cum, (32, E))).astype(jnp.int32)
    eotc = jnp.sum(cmp, axis=1, keepdims=True)
    eot_ref[...] = jnp.broadcast_to(eotc, (32, 128))

    hi = jax.lax.Precision.HIGHEST
    b1 = jnp.dot(base, w8_ref[...], preferred_element_type=jnp.float32,
                 precision=hi)
    b2 = jnp.dot(base + tot1, w8_ref[...], preferred_element_type=jnp.float32,
                 precision=hi)
    o1 = jnp.dot(off1, w8_ref[...], preferred_element_type=jnp.float32,
                 precision=hi)
    o2 = jnp.dot(off2, w8_ref[...], preferred_element_type=jnp.float32,
                 precision=hi)
    x1 = a1 * (c1 - 1.0 + o1 + jnp.broadcast_to(b1, (G, E * 128)))
    x2 = a2 * (c2 - 1.0 + o2 + jnp.broadcast_to(b2, (G, E * 128)))
    p1 = jnp.dot(x1, qred_ref[...], preferred_element_type=jnp.float32,
                 precision=hi)
    p2 = jnp.dot(x2, qred_ref[...], preferred_element_type=jnp.float32,
                 precision=hi)
    pos1_ref[...] = p1.astype(jnp.int32)
    pos2_ref[...] = p2.astype(jnp.int32)


def _sort_consts():
    l = np.arange(128)
    e = np.arange(E)
    c_l, c_e = np.divmod(np.arange(E * 128), E)
    qt = (l[:, None] == c_l[None, :]).astype(np.float32)
    u = ((c_e[:, None] == c_e[None, :])
         & (c_l[:, None] <= c_l[None, :])).astype(np.float32)
    qred = (c_l[:, None] == l[None, :]).astype(np.float32)
    w8 = (e[:, None] == c_e[None, :]).astype(np.float32)
    l16 = (np.arange(G)[None, :] < np.arange(G)[:, None]).astype(np.float32)
    mlt = (e[:, None] < e[None, :]).astype(np.float32)
    return (jnp.asarray(u, jnp.bfloat16), jnp.asarray(qt),
            jnp.asarray(qred), jnp.asarray(w8), jnp.asarray(l16),
            jnp.asarray(mlt))


def _routing(x, w_gate):
    u, qt, qred, w8, l16, mlt = _sort_consts()
    full = lambda s: pl.BlockSpec(s, lambda: (0,) * len(s))
    outs = pl.pallas_call(
        _routing_kernel,
        in_specs=[full((T, D)), full((D, E)), full((E * 128, E * 128)),
                  full((128, E * 128)), full((E * 128, 128)),
                  full((E, E * 128)), full((G, G)), full((E, E))],
        out_specs=[full((G, 128)), full((G, 128)), full((T, 16)),
                   full((T, 16)), full((32, 128))],
        out_shape=[
            jax.ShapeDtypeStruct((G, 128), jnp.int32),
            jax.ShapeDtypeStruct((G, 128), jnp.int32),
            jax.ShapeDtypeStruct((T, 16), jnp.float32),
            jax.ShapeDtypeStruct((T, 16), jnp.float32),
            jax.ShapeDtypeStruct((32, 128), jnp.int32),
        ],
    )(x, w_gate, u, qt, qred, w8, l16, mlt)
    return outs


@functools.cache
def _make_dispatch():
    mesh = plsc.VectorSubcoreMesh(core_axis_name="c", subcore_axis_name="s")

    @functools.partial(
        pl.kernel, mesh=mesh,
        out_type=jax.ShapeDtypeStruct((SP, D), jnp.float32),
        scratch_types=[
            pltpu.VMEM((CHUNK,), jnp.int32),
            pltpu.VMEM((CHUNK,), jnp.int32),
            pltpu.VMEM((CHUNK, D), jnp.float32),
            pltpu.SemaphoreType.DMA,
        ],
    )
    def _dispatch(x_hbm, p1_hbm, p2_hbm, xg_hbm, i1_v, i2_v, rows_v, sem):
        wid = jax.lax.axis_index("s") * 2 + jax.lax.axis_index("c")
        base = wid * CHUNK
        c1 = pltpu.async_copy(p1_hbm.at[pl.ds(base, CHUNK)], i1_v, sem)
        c2 = pltpu.async_copy(p2_hbm.at[pl.ds(base, CHUNK)], i2_v, sem)
        c3 = pltpu.async_copy(x_hbm.at[pl.ds(base, CHUNK)], rows_v, sem)
        c1.wait()
        c2.wait()
        c3.wait()
        s1 = pltpu.async_copy(rows_v, xg_hbm.at[i1_v], sem)
        s2 = pltpu.async_copy(rows_v, xg_hbm.at[i2_v], sem)
        s1.wait()
        s2.wait()

    return _dispatch


def _gmm_kernel(eot_ref, xg_ref, wgu_ref, wd_ref, yg_ref):
    e = eot_ref[pl.program_id(0)]

    @pl.when(e < E)
    def _():
        xb = xg_ref[...].astype(jnp.bfloat16)
        gu = jnp.dot(xb, wgu_ref[0], preferred_element_type=jnp.float32)
        g = gu[:, :I]
        u = gu[:, I:]
        h = (g * jax.lax.logistic(g)) * u
        yg_ref[...] = jnp.dot(h.astype(jnp.bfloat16), wd_ref[0],
                              preferred_element_type=jnp.float32)


def _gmm(eot, xg, wgu, wd):
    wexp = lambda i, eot_ref: (jnp.minimum(eot_ref[i], E - 1), 0, 0)
    grid_spec = pltpu.PrefetchScalarGridSpec(
        num_scalar_prefetch=1,
        grid=(NT,),
        in_specs=[
            pl.BlockSpec((TR, D), lambda i, eot_ref: (i, 0)),
            pl.BlockSpec((1, D, 2 * I), wexp),
            pl.BlockSpec((1, I, D), wexp),
        ],
        out_specs=pl.BlockSpec((TR, D), lambda i, eot_ref: (i, 0)),
    )
    return pl.pallas_call(
        _gmm_kernel,
        grid_spec=grid_spec,
        out_shape=jax.ShapeDtypeStruct((SP, D), jnp.float32),
        compiler_params=pltpu.CompilerParams(
            dimension_semantics=("arbitrary",),
        ),
    )(eot, xg, wgu, wd)


def _shared_kernel(x_ref, sgu_ref, sd_ref, o_ref):
    xb = x_ref[...].astype(jnp.bfloat16)
    gu = jnp.dot(xb, sgu_ref[...], preferred_element_type=jnp.float32)
    g = gu[:, :I]
    u = gu[:, I:]
    h = (g * jax.lax.logistic(g)) * u
    o_ref[...] = jnp.dot(h.astype(jnp.bfloat16), sd_ref[...],
                         preferred_element_type=jnp.float32)


def _shared(x, sgu, sd):
    return pl.pallas_call(
        _shared_kernel,
        grid=(2,),
        in_specs=[
            pl.BlockSpec((T // 2, D), lambda m: (m, 0)),
            pl.BlockSpec((D, 2 * I), lambda m: (0, 0)),
            pl.BlockSpec((I, D), lambda m: (0, 0)),
        ],
        out_specs=pl.BlockSpec((T // 2, D), lambda m: (m, 0)),
        out_shape=jax.ShapeDtypeStruct((T, D), jnp.float32),
    )(x, sgu, sd)


@functools.cache
def _make_combine():
    mesh = plsc.VectorSubcoreMesh(core_axis_name="c", subcore_axis_name="s")

    @functools.partial(
        pl.kernel, mesh=mesh,
        out_type=jax.ShapeDtypeStruct((T, D), jnp.float32),
        scratch_types=[
            pltpu.VMEM((CB,), jnp.int32),
            pltpu.VMEM((CB,), jnp.int32),
            pltpu.VMEM((CB, D), jnp.float32),
            pltpu.VMEM((CB, D), jnp.float32),
            pltpu.VMEM((CB, D), jnp.float32),
            pltpu.VMEM((CB, 16), jnp.float32),
            pltpu.VMEM((CB, 16), jnp.float32),
            pltpu.SemaphoreType.DMA,
        ],
    )
    def _combine(yg_hbm, sh_hbm, p1_hbm, p2_hbm, w1_hbm, w2_hbm, out_hbm,
                 i1_v, i2_v, y1_v, y2_v, acc_v, w1_v, w2_v, sem):
        wid = jax.lax.axis_index("s") * 2 + jax.lax.axis_index("c")
        for b in range(CHUNK // CB):
            base = wid * CHUNK + b * CB
            c1 = pltpu.async_copy(p1_hbm.at[pl.ds(base, CB)], i1_v, sem)
            c2 = pltpu.async_copy(p2_hbm.at[pl.ds(base, CB)], i2_v, sem)
            c3 = pltpu.async_copy(sh_hbm.at[pl.ds(base, CB)], acc_v, sem)
            c4 = pltpu.async_copy(w1_hbm.at[pl.ds(base, CB)], w1_v, sem)
            c5 = pltpu.async_copy(w2_hbm.at[pl.ds(base, CB)], w2_v, sem)
            c1.wait()
            c2.wait()
            g1 = pltpu.async_copy(yg_hbm.at[i1_v], y1_v, sem)
            g2 = pltpu.async_copy(yg_hbm.at[i2_v], y2_v, sem)
            c3.wait()
            c4.wait()
            c5.wait()
            g1.wait()
            g2.wait()

            def body(t, carry):
                w1t = w1_v[t, :]
                w2t = w2_v[t, :]
                for j in range(D // 16):
                    sl = pl.ds(j * 16, 16)
                    acc_v[t, sl] = (acc_v[t, sl] + w1t * y1_v[t, sl]
                                    + w2t * y2_v[t, sl])
                return carry

            jax.lax.fori_loop(0, CB, body, 0)
            pltpu.sync_copy(acc_v, out_hbm.at[pl.ds(base, CB)])

    return _combine


def kernel(hidden_states, w_gate, w_gate_up, w_down, shared_gate_up,
           shared_down):
    x = hidden_states
    wgu = w_gate_up.astype(jnp.bfloat16)
    wd = w_down.astype(jnp.bfloat16)
    sgu = shared_gate_up.astype(jnp.bfloat16)
    sd = shared_down.astype(jnp.bfloat16)

    pos1_2d, pos2_2d, w1rep, w2rep, eot2d = _routing(x, w_gate)
    pos1 = pos1_2d.reshape(T)
    pos2 = pos2_2d.reshape(T)
    eot = eot2d[:NT, 0]

    xg = _make_dispatch()(x, pos1, pos2)
    sh = _shared(x, sgu, sd)
    yg = _gmm(eot, xg, wgu, wd)
    out = _make_combine()(yg, sh, pos1, pos2, w1rep, w2rep)
    return out
